# Initial kernel scaffold; baseline (speedup 1.0000x reference)
import jax
import jax.numpy as jnp
from jax.experimental import pallas as pl


def kernel(token_ids, table):
    raise NotImplementedError("write your pallas kernel here")


# speedup vs baseline: 5.1029x; 5.1029x over previous
import functools
import numpy as np
import jax
import jax.numpy as jnp
from jax import lax
from jax.experimental import pallas as pl
from jax.experimental.pallas import tpu as pltpu
from jax.experimental.pallas import tpu_sc as plsc

LANES = 16
NUM_WORKERS = 32


def _sinusoidal_pe_np(seq_len, d_model):
    pos = np.arange(seq_len, dtype=np.float32)[:, None]
    div = np.exp(
        np.arange(0, d_model, 2, dtype=np.float32) * (-np.log(10000.0) / d_model)
    )
    pe = np.zeros((seq_len, d_model), dtype=np.float32)
    pe[:, 0::2] = np.sin(pos * div)
    pe[:, 1::2] = np.cos(pos * div)
    return pe


@functools.partial(jax.jit, static_argnames=("n", "s", "d"))
def _embed_pe(table, ids_flat, pe, *, n, s, d):
    mesh = plsc.VectorSubcoreMesh(core_axis_name="c", subcore_axis_name="s")
    n_per_w = n // NUM_WORKERS
    nseq = n_per_w // s
    h0 = 96
    h1 = s - h0

    @functools.partial(
        pl.kernel,
        out_type=jax.ShapeDtypeStruct((n, d), jnp.float32),
        mesh=mesh,
        scratch_types=[
            pltpu.VMEM((s, d), jnp.float32),
            pltpu.VMEM((n_per_w,), jnp.int32),
            pltpu.VMEM((s, d), jnp.float32),
            pltpu.VMEM((s, d), jnp.float32),
            pltpu.SemaphoreType.DMA,
            pltpu.SemaphoreType.DMA,
            pltpu.SemaphoreType.DMA,
            pltpu.SemaphoreType.DMA,
        ],
    )
    def k(table_hbm, idx_hbm, pe_hbm, out_hbm, pe_v, idx_v, buf0, buf1, g0, g1, s0, s1):
        wid = lax.axis_index("s") * 2 + lax.axis_index("c")
        base = wid * n_per_w
        pltpu.sync_copy(pe_hbm, pe_v)
        pltpu.sync_copy(idx_hbm.at[pl.ds(base, n_per_w)], idx_v)

        def gather_seq(j, buf, sem):
            c0 = pltpu.async_copy(
                table_hbm.at[idx_v.at[pl.ds(j * s, h0)]],
                buf.at[pl.ds(0, h0)],
                sem,
            )
            c1 = pltpu.async_copy(
                table_hbm.at[idx_v.at[pl.ds(j * s + h0, h1)]],
                buf.at[pl.ds(h0, h1)],
                sem,
            )
            return c0, c1

        def add_pe(buf):
            @pl.loop(0, s)
            def _(r):
                for c in range(d // LANES):
                    slc = (pl.ds(r, 1), pl.ds(c * LANES, LANES))
                    buf.at[*slc][...] = buf.at[*slc][...] + pe_v.at[*slc][...]

        @pl.loop(0, nseq, step=2)
        def _(j):
            ga = gather_seq(j, buf0, g0)
            gb = gather_seq(j + 1, buf1, g1)
            ga[0].wait()
            ga[1].wait()
            add_pe(buf0)
            sa = pltpu.async_copy(buf0, out_hbm.at[pl.ds(base + j * s, s)], s0)
            gb[0].wait()
            gb[1].wait()
            add_pe(buf1)
            sb = pltpu.async_copy(buf1, out_hbm.at[pl.ds(base + (j + 1) * s, s)], s1)
            sa.wait()
            sb.wait()

    return k(table, ids_flat, pe)


def kernel(token_ids, table):
    b, s = token_ids.shape
    v, d = table.shape
    ids_flat = token_ids.reshape(b * s).astype(jnp.int32)
    pe = jnp.asarray(_sinusoidal_pe_np(s, d))
    out = _embed_pe(table, ids_flat, pe, n=b * s, s=s, d=d)
    return out.reshape(b, s, d)
